# baseline (speedup 1.0000x reference)
import functools

import jax
import jax.numpy as jnp
from jax import lax
from jax.experimental import pallas as pl
from jax.experimental.pallas import tpu as pltpu
from jax.experimental.pallas import tpu_sc as plsc

_NC = 2
_NS = 16
_NW = _NC * _NS

_CHUNK = 64
_NBUF = 4
_IBLK = 16


def _sc_mesh():
    return plsc.VectorSubcoreMesh(core_axis_name="c", subcore_axis_name="s")


def _make_deg_kernel(n_pad, e_pad):
    nch = e_pad // _CHUNK // _NW
    rs = n_pad // _NS
    k = 16

    @functools.partial(
        pl.kernel,
        out_type=jax.ShapeDtypeStruct((_NC * n_pad,), jnp.float32),
        mesh=_sc_mesh(),
        scratch_types=[
            pltpu.VMEM((nch, _CHUNK), jnp.int32),
            pltpu.VMEM((_CHUNK,), jnp.float32),
            pltpu.VMEM_SHARED((n_pad,), jnp.float32),
            pltpu.SemaphoreType.DMA,
        ],
    )
    def deg_kernel(dst_hbm, zeros_hbm, ones_hbm, deg_out, didx, ones_v, deg_sh, sem):
        c = lax.axis_index("c")
        s = lax.axis_index("s")
        wid = s * _NC + c
        pltpu.sync_copy(ones_hbm, ones_v)
        pltpu.sync_copy(dst_hbm.at[pl.ds(wid * nch, nch)], didx)
        pltpu.sync_copy(zeros_hbm.at[pl.ds(s * rs, rs)], deg_sh.at[pl.ds(s * rs, rs)])
        plsc.subcore_barrier()

        def group(gi, carry):
            for b in range(k):
                pltpu.async_copy(ones_v, deg_sh.at[didx.at[gi * k + b]], sem, add=True)
            for b in range(k):
                pltpu.make_async_copy(ones_v, deg_sh.at[didx.at[0]], sem).wait()
            return carry

        lax.fori_loop(0, nch // k, group, 0)
        plsc.subcore_barrier()
        pltpu.sync_copy(deg_sh.at[pl.ds(s * rs, rs)],
                        deg_out.at[pl.ds(c * n_pad + s * rs, rs)])

    return deg_kernel


def _make_edge_kernel(n_pad, e_pad, d):
    nch = e_pad // _CHUNK // _NW
    rs = n_pad // _NS
    ng = nch // _IBLK
    nz = rs // _CHUNK

    @functools.partial(
        pl.kernel,
        out_type=jax.ShapeDtypeStruct((_NC * n_pad, d), jnp.float32),
        mesh=_sc_mesh(),
        scratch_types=[
            pltpu.VMEM((2 * _IBLK, _CHUNK), jnp.int32),
            pltpu.VMEM((2 * _IBLK, _CHUNK), jnp.int32),
        ] + [pltpu.VMEM((_CHUNK, d), jnp.float32)] * _NBUF + [
            pltpu.VMEM_SHARED((n_pad, d), jnp.float32),
        ] + [pltpu.SemaphoreType.DMA] * (2 * _NBUF + 1),
    )
    def edge_kernel(y_hbm, src_hbm, dst_hbm, zblk_hbm, acc_out,
                    sidx, didx, *rest):
        rows = rest[:_NBUF]
        acc_sh = rest[_NBUF]
        sems = rest[_NBUF + 1:]
        gsem = sems[:_NBUF]
        ssem = sems[_NBUF:2 * _NBUF]
        isem = sems[2 * _NBUF]
        c = lax.axis_index("c")
        s = lax.axis_index("s")
        wid = s * _NC + c
        base = wid * nch
        pltpu.sync_copy(src_hbm.at[pl.ds(base, _IBLK)], sidx.at[pl.ds(0, _IBLK)])
        pltpu.sync_copy(dst_hbm.at[pl.ds(base, _IBLK)], didx.at[pl.ds(0, _IBLK)])
        pltpu.sync_copy(zblk_hbm, rows[0])
        for z in range(nz):
            pltpu.sync_copy(rows[0], acc_sh.at[pl.ds(s * rs + z * _CHUNK, _CHUNK)])
        plsc.subcore_barrier()

        def group(g, carry):
            off = lax.rem(g, 2) * _IBLK
            noff = lax.rem(g + 1, 2) * _IBLK
            nxt = base + jnp.minimum(g + 1, ng - 1) * _IBLK
            pltpu.async_copy(src_hbm.at[pl.ds(nxt, _IBLK)],
                             sidx.at[pl.ds(noff, _IBLK)], isem)
            pltpu.async_copy(dst_hbm.at[pl.ds(nxt, _IBLK)],
                             didx.at[pl.ds(noff, _IBLK)], isem)

            def fire_gather(j):
                pltpu.async_copy(y_hbm.at[sidx.at[off + j]],
                                 rows[j % _NBUF], gsem[j % _NBUF])

            def wait_gather(j):
                pltpu.make_async_copy(y_hbm.at[sidx.at[0]],
                                      rows[j % _NBUF], gsem[j % _NBUF]).wait()

            def fire_scatter(j):
                pltpu.async_copy(rows[j % _NBUF], acc_sh.at[didx.at[off + j]],
                                 ssem[j % _NBUF], add=True)

            def wait_scatter(j):
                pltpu.make_async_copy(rows[j % _NBUF], acc_sh.at[didx.at[0]],
                                      ssem[j % _NBUF]).wait()

            for j in range(_NBUF):
                fire_gather(j)
            for j in range(_IBLK):
                wait_gather(j)
                fire_scatter(j)
                if j + _NBUF < _IBLK:
                    wait_scatter(j)
                    fire_gather(j + _NBUF)
                else:
                    wait_scatter(j)
            pltpu.make_async_copy(src_hbm.at[pl.ds(0, _IBLK)],
                                  sidx.at[pl.ds(0, _IBLK)], isem).wait()
            pltpu.make_async_copy(dst_hbm.at[pl.ds(0, _IBLK)],
                                  didx.at[pl.ds(0, _IBLK)], isem).wait()
            return carry

        lax.fori_loop(0, ng, group, 0)
        plsc.subcore_barrier()
        pltpu.sync_copy(acc_sh.at[pl.ds(s * rs, rs)],
                        acc_out.at[pl.ds(c * n_pad + s * rs, rs)])

    return edge_kernel


def _mm_scale_body(x_ref, w_ref, dega_ref, degb_ref, dinv_ref, y_ref):
    deg = dega_ref[...] + degb_ref[...] + 1.0
    dinv = lax.rsqrt(jnp.maximum(deg, 1.0))
    dinv_ref[...] = dinv
    xw = jnp.dot(x_ref[...], w_ref[...], preferred_element_type=jnp.float32)
    y_ref[...] = xw * dinv[:, None]


def _layer2_body(acca_ref, accb_ref, y1_ref, dinv_ref, b1_ref, w2_ref, y2_ref):
    dinv = dinv_ref[...]
    h = dinv[:, None] * (acca_ref[...] + accb_ref[...] + y1_ref[...]) + b1_ref[...][None, :]
    h = jnp.maximum(h, 0.0)
    y2_ref[...] = jnp.dot(h, w2_ref[...], preferred_element_type=jnp.float32) * dinv[:, None]


def _pool_body(g, acca_ref, accb_ref, y2_ref, dinv_ref, b2_ref, batch_ref,
               out_ref, acc_s, cnt_s):
    i = pl.program_id(0)

    @pl.when(i == 0)
    def _init():
        acc_s[...] = jnp.zeros_like(acc_s)
        cnt_s[...] = jnp.zeros_like(cnt_s)

    dinv = dinv_ref[...]
    o2 = dinv[:, None] * (acca_ref[...] + accb_ref[...] + y2_ref[...]) + b2_ref[...][None, :]
    onehot = (batch_ref[...][:, None]
              == lax.broadcasted_iota(jnp.int32, (1, g), 1)).astype(jnp.float32)
    acc_s[...] += lax.dot_general(onehot, o2, (((0,), (0,)), ((), ())),
                                  preferred_element_type=jnp.float32)
    cnt_s[...] += jnp.sum(onehot, axis=0)

    @pl.when(i == pl.num_programs(0) - 1)
    def _fin():
        out_ref[...] = acc_s[...] / jnp.maximum(cnt_s[...], 1.0)[:, None]


def kernel(x, adj_t, batch, W1, b1, W2, b2):
    n, d = x.shape
    e = adj_t.shape[1]
    h2 = W2.shape[1]
    g = 64

    n_pad = -(-(n + 64) // 1024) * 1024
    e_pad = -(-e // (_NW * _CHUNK * _IBLK)) * (_NW * _CHUNK * _IBLK)
    padrows = n_pad - n
    pe = e_pad - e

    src = adj_t[0]
    dst = adj_t[1]
    pad_idx = (n + (jnp.arange(pe, dtype=jnp.int32) % jnp.int32(padrows)))
    src_p = jnp.concatenate([src, pad_idx]).reshape(e_pad // _CHUNK, _CHUNK)
    dst_p = jnp.concatenate([dst, pad_idx]).reshape(e_pad // _CHUNK, _CHUNK)
    x_p = jnp.pad(x, ((0, n_pad - n), (0, 0)))
    batch_p = jnp.pad(batch, (0, n_pad - n), constant_values=g)

    zeros_vec = jnp.zeros((n_pad,), jnp.float32)
    zeros_blk = jnp.zeros((_CHUNK, d), jnp.float32)
    ones_vec = jnp.ones((_CHUNK,), jnp.float32)

    deg_kernel = _make_deg_kernel(n_pad, e_pad)
    edge_kernel = _make_edge_kernel(n_pad, e_pad, d)

    rb = 1024
    nblk = n_pad // rb

    deg2 = deg_kernel(dst_p, zeros_vec, ones_vec)

    dinv, y1 = pl.pallas_call(
        _mm_scale_body,
        grid=(nblk,),
        in_specs=[
            pl.BlockSpec((rb, d), lambda i: (i, 0)),
            pl.BlockSpec((d, d), lambda i: (0, 0)),
            pl.BlockSpec((rb,), lambda i: (i,)),
            pl.BlockSpec((rb,), lambda i: (i + nblk,)),
        ],
        out_specs=[
            pl.BlockSpec((rb,), lambda i: (i,)),
            pl.BlockSpec((rb, d), lambda i: (i, 0)),
        ],
        out_shape=[
            jax.ShapeDtypeStruct((n_pad,), jnp.float32),
            jax.ShapeDtypeStruct((n_pad, d), jnp.float32),
        ],
    )(x_p, W1, deg2, deg2)

    acc1 = edge_kernel(y1, src_p, dst_p, zeros_blk)

    y2 = pl.pallas_call(
        _layer2_body,
        grid=(nblk,),
        in_specs=[
            pl.BlockSpec((rb, d), lambda i: (i, 0)),
            pl.BlockSpec((rb, d), lambda i: (i + nblk, 0)),
            pl.BlockSpec((rb, d), lambda i: (i, 0)),
            pl.BlockSpec((rb,), lambda i: (i,)),
            pl.BlockSpec((d,), lambda i: (0,)),
            pl.BlockSpec((d, h2), lambda i: (0, 0)),
        ],
        out_specs=pl.BlockSpec((rb, h2), lambda i: (i, 0)),
        out_shape=jax.ShapeDtypeStruct((n_pad, h2), jnp.float32),
    )(acc1, acc1, y1, dinv, b1, W2)

    acc2 = edge_kernel(y2, src_p, dst_p, zeros_blk)

    out = pl.pallas_call(
        functools.partial(_pool_body, g),
        grid=(nblk,),
        in_specs=[
            pl.BlockSpec((rb, h2), lambda i: (i, 0)),
            pl.BlockSpec((rb, h2), lambda i: (i + nblk, 0)),
            pl.BlockSpec((rb, h2), lambda i: (i, 0)),
            pl.BlockSpec((rb,), lambda i: (i,)),
            pl.BlockSpec((h2,), lambda i: (0,)),
            pl.BlockSpec((rb,), lambda i: (i,)),
        ],
        out_specs=pl.BlockSpec((g, h2), lambda i: (0, 0)),
        out_shape=jax.ShapeDtypeStruct((g, h2), jnp.float32),
        scratch_shapes=[
            pltpu.VMEM((g, h2), jnp.float32),
            pltpu.VMEM((g,), jnp.float32),
        ],
    )(acc2, acc2, y2, dinv, b2, batch_p)

    return out


# speedup vs baseline: 1.1232x; 1.1232x over previous
import functools

import jax
import jax.numpy as jnp
from jax import lax
from jax.experimental import pallas as pl
from jax.experimental.pallas import tpu as pltpu
from jax.experimental.pallas import tpu_sc as plsc

_NC = 2
_NS = 16
_NW = _NC * _NS

_CHUNK = 64
_NBUF = 4
_IBLK = 16


def _sc_mesh():
    return plsc.VectorSubcoreMesh(core_axis_name="c", subcore_axis_name="s")


def _make_deg_kernel(n_pad, e_pad):
    nch = e_pad // _CHUNK // _NW
    rs = n_pad // _NS
    k = 16

    @functools.partial(
        pl.kernel,
        out_type=jax.ShapeDtypeStruct((_NC * n_pad,), jnp.float32),
        mesh=_sc_mesh(),
        scratch_types=[
            pltpu.VMEM((nch, _CHUNK), jnp.int32),
            pltpu.VMEM((_CHUNK,), jnp.float32),
            pltpu.VMEM_SHARED((n_pad,), jnp.float32),
            pltpu.SemaphoreType.DMA,
        ],
    )
    def deg_kernel(dst_hbm, zeros_hbm, ones_hbm, deg_out, didx, ones_v, deg_sh, sem):
        c = lax.axis_index("c")
        s = lax.axis_index("s")
        wid = s * _NC + c
        pltpu.sync_copy(ones_hbm, ones_v)
        pltpu.sync_copy(dst_hbm.at[pl.ds(wid * nch, nch)], didx)
        pltpu.sync_copy(zeros_hbm.at[pl.ds(s * rs, rs)], deg_sh.at[pl.ds(s * rs, rs)])
        plsc.subcore_barrier()

        def group(gi, carry):
            for b in range(k):
                pltpu.async_copy(ones_v, deg_sh.at[didx.at[gi * k + b]], sem, add=True)
            for b in range(k):
                pltpu.make_async_copy(ones_v, deg_sh.at[didx.at[0]], sem).wait()
            return carry

        lax.fori_loop(0, nch // k, group, 0)
        plsc.subcore_barrier()
        pltpu.sync_copy(deg_sh.at[pl.ds(s * rs, rs)],
                        deg_out.at[pl.ds(c * n_pad + s * rs, rs)])

    return deg_kernel


def _make_edge_kernel(n_pad, e_pad, d):
    nch = e_pad // _CHUNK // _NW
    rs = n_pad // _NS
    ng = nch // _IBLK
    nz = rs // _CHUNK

    @functools.partial(
        pl.kernel,
        out_type=jax.ShapeDtypeStruct((_NC * n_pad, d), jnp.float32),
        mesh=_sc_mesh(),
        scratch_types=[
            pltpu.VMEM((2 * _IBLK, _CHUNK), jnp.int32),
            pltpu.VMEM((2 * _IBLK, _CHUNK), jnp.int32),
        ] + [pltpu.VMEM((_CHUNK, d), jnp.float32)] * _NBUF + [
            pltpu.VMEM_SHARED((n_pad, d), jnp.float32),
        ] + [pltpu.SemaphoreType.DMA] * (2 * _NBUF + 1),
    )
    def edge_kernel(y_hbm, src_hbm, dst_hbm, zblk_hbm, acc_out,
                    sidx, didx, *rest):
        rows = rest[:_NBUF]
        acc_sh = rest[_NBUF]
        sems = rest[_NBUF + 1:]
        gsem = sems[:_NBUF]
        ssem = sems[_NBUF:2 * _NBUF]
        isem = sems[2 * _NBUF]
        c = lax.axis_index("c")
        s = lax.axis_index("s")
        wid = s * _NC + c
        base = wid * nch
        pltpu.sync_copy(src_hbm.at[pl.ds(base, _IBLK)], sidx.at[pl.ds(0, _IBLK)])
        pltpu.sync_copy(dst_hbm.at[pl.ds(base, _IBLK)], didx.at[pl.ds(0, _IBLK)])
        pltpu.sync_copy(zblk_hbm, rows[0])
        for z in range(nz):
            pltpu.sync_copy(rows[0], acc_sh.at[pl.ds(s * rs + z * _CHUNK, _CHUNK)])
        plsc.subcore_barrier()

        def fire_gather(off, j):
            pltpu.async_copy(y_hbm.at[sidx.at[off + j]],
                             rows[j % _NBUF], gsem[j % _NBUF])

        def wait_gather(j):
            pltpu.make_async_copy(y_hbm.at[sidx.at[0]],
                                  rows[j % _NBUF], gsem[j % _NBUF]).wait()

        def fire_scatter(off, j):
            pltpu.async_copy(rows[j % _NBUF], acc_sh.at[didx.at[off + j]],
                             ssem[j % _NBUF], add=True)

        def wait_scatter(j):
            pltpu.make_async_copy(rows[j % _NBUF], acc_sh.at[didx.at[0]],
                                  ssem[j % _NBUF]).wait()

        for j in range(_NBUF):
            fire_gather(0, j)

        def group(g, carry):
            off = lax.rem(g, 2) * _IBLK
            noff = lax.rem(g + 1, 2) * _IBLK
            nxt = base + (g + 1) * _IBLK
            pltpu.async_copy(src_hbm.at[pl.ds(nxt, _IBLK)],
                             sidx.at[pl.ds(noff, _IBLK)], isem)
            pltpu.async_copy(dst_hbm.at[pl.ds(nxt, _IBLK)],
                             didx.at[pl.ds(noff, _IBLK)], isem)
            for j in range(_IBLK):
                wait_gather(j)
                fire_scatter(off, j)
                wait_scatter(j)
                if j + _NBUF < _IBLK:
                    fire_gather(off, j + _NBUF)
                else:
                    if j + _NBUF == _IBLK:
                        pltpu.make_async_copy(src_hbm.at[pl.ds(0, _IBLK)],
                                              sidx.at[pl.ds(0, _IBLK)],
                                              isem).wait()
                        pltpu.make_async_copy(dst_hbm.at[pl.ds(0, _IBLK)],
                                              didx.at[pl.ds(0, _IBLK)],
                                              isem).wait()
                    fire_gather(noff, j + _NBUF - _IBLK)
            return carry

        lax.fori_loop(0, ng - 1, group, 0)
        loff = lax.rem(jnp.int32(ng - 1), 2) * _IBLK
        for j in range(_IBLK):
            wait_gather(j)
            fire_scatter(loff, j)
            wait_scatter(j)
            if j + _NBUF < _IBLK:
                fire_gather(loff, j + _NBUF)
        plsc.subcore_barrier()
        pltpu.sync_copy(acc_sh.at[pl.ds(s * rs, rs)],
                        acc_out.at[pl.ds(c * n_pad + s * rs, rs)])

    return edge_kernel


def _mm_scale_body(x_ref, w_ref, dega_ref, degb_ref, dinv_ref, y_ref):
    deg = dega_ref[...] + degb_ref[...] + 1.0
    dinv = lax.rsqrt(jnp.maximum(deg, 1.0))
    dinv_ref[...] = dinv
    xw = jnp.dot(x_ref[...], w_ref[...], preferred_element_type=jnp.float32)
    y_ref[...] = xw * dinv[:, None]


def _layer2_body(acca_ref, accb_ref, y1_ref, dinv_ref, b1_ref, w2_ref, y2_ref):
    dinv = dinv_ref[...]
    h = dinv[:, None] * (acca_ref[...] + accb_ref[...] + y1_ref[...]) + b1_ref[...][None, :]
    h = jnp.maximum(h, 0.0)
    y2_ref[...] = jnp.dot(h, w2_ref[...], preferred_element_type=jnp.float32) * dinv[:, None]


def _pool_body(g, acca_ref, accb_ref, y2_ref, dinv_ref, b2_ref, batch_ref,
               out_ref, acc_s, cnt_s):
    i = pl.program_id(0)

    @pl.when(i == 0)
    def _init():
        acc_s[...] = jnp.zeros_like(acc_s)
        cnt_s[...] = jnp.zeros_like(cnt_s)

    dinv = dinv_ref[...]
    o2 = dinv[:, None] * (acca_ref[...] + accb_ref[...] + y2_ref[...]) + b2_ref[...][None, :]
    onehot = (batch_ref[...][:, None]
              == lax.broadcasted_iota(jnp.int32, (1, g), 1)).astype(jnp.float32)
    acc_s[...] += lax.dot_general(onehot, o2, (((0,), (0,)), ((), ())),
                                  preferred_element_type=jnp.float32)
    cnt_s[...] += jnp.sum(onehot, axis=0)

    @pl.when(i == pl.num_programs(0) - 1)
    def _fin():
        out_ref[...] = acc_s[...] / jnp.maximum(cnt_s[...], 1.0)[:, None]


def kernel(x, adj_t, batch, W1, b1, W2, b2):
    n, d = x.shape
    e = adj_t.shape[1]
    h2 = W2.shape[1]
    g = 64

    n_pad = -(-(n + 64) // 1024) * 1024
    e_pad = -(-e // (_NW * _CHUNK * _IBLK)) * (_NW * _CHUNK * _IBLK)
    padrows = n_pad - n
    pe = e_pad - e

    src = adj_t[0]
    dst = adj_t[1]
    pad_idx = (n + (jnp.arange(pe, dtype=jnp.int32) % jnp.int32(padrows)))
    src_p = jnp.concatenate([src, pad_idx]).reshape(e_pad // _CHUNK, _CHUNK)
    dst_p = jnp.concatenate([dst, pad_idx]).reshape(e_pad // _CHUNK, _CHUNK)
    x_p = jnp.pad(x, ((0, n_pad - n), (0, 0)))
    batch_p = jnp.pad(batch, (0, n_pad - n), constant_values=g)

    zeros_vec = jnp.zeros((n_pad,), jnp.float32)
    zeros_blk = jnp.zeros((_CHUNK, d), jnp.float32)
    ones_vec = jnp.ones((_CHUNK,), jnp.float32)

    deg_kernel = _make_deg_kernel(n_pad, e_pad)
    edge_kernel = _make_edge_kernel(n_pad, e_pad, d)

    rb = 1024
    nblk = n_pad // rb

    deg2 = deg_kernel(dst_p, zeros_vec, ones_vec)

    dinv, y1 = pl.pallas_call(
        _mm_scale_body,
        grid=(nblk,),
        in_specs=[
            pl.BlockSpec((rb, d), lambda i: (i, 0)),
            pl.BlockSpec((d, d), lambda i: (0, 0)),
            pl.BlockSpec((rb,), lambda i: (i,)),
            pl.BlockSpec((rb,), lambda i: (i + nblk,)),
        ],
        out_specs=[
            pl.BlockSpec((rb,), lambda i: (i,)),
            pl.BlockSpec((rb, d), lambda i: (i, 0)),
        ],
        out_shape=[
            jax.ShapeDtypeStruct((n_pad,), jnp.float32),
            jax.ShapeDtypeStruct((n_pad, d), jnp.float32),
        ],
    )(x_p, W1, deg2, deg2)

    acc1 = edge_kernel(y1, src_p, dst_p, zeros_blk)

    y2 = pl.pallas_call(
        _layer2_body,
        grid=(nblk,),
        in_specs=[
            pl.BlockSpec((rb, d), lambda i: (i, 0)),
            pl.BlockSpec((rb, d), lambda i: (i + nblk, 0)),
            pl.BlockSpec((rb, d), lambda i: (i, 0)),
            pl.BlockSpec((rb,), lambda i: (i,)),
            pl.BlockSpec((d,), lambda i: (0,)),
            pl.BlockSpec((d, h2), lambda i: (0, 0)),
        ],
        out_specs=pl.BlockSpec((rb, h2), lambda i: (i, 0)),
        out_shape=jax.ShapeDtypeStruct((n_pad, h2), jnp.float32),
    )(acc1, acc1, y1, dinv, b1, W2)

    acc2 = edge_kernel(y2, src_p, dst_p, zeros_blk)

    out = pl.pallas_call(
        functools.partial(_pool_body, g),
        grid=(nblk,),
        in_specs=[
            pl.BlockSpec((rb, h2), lambda i: (i, 0)),
            pl.BlockSpec((rb, h2), lambda i: (i + nblk, 0)),
            pl.BlockSpec((rb, h2), lambda i: (i, 0)),
            pl.BlockSpec((rb,), lambda i: (i,)),
            pl.BlockSpec((h2,), lambda i: (0,)),
            pl.BlockSpec((rb,), lambda i: (i,)),
        ],
        out_specs=pl.BlockSpec((g, h2), lambda i: (0, 0)),
        out_shape=jax.ShapeDtypeStruct((g, h2), jnp.float32),
        scratch_shapes=[
            pltpu.VMEM((g, h2), jnp.float32),
            pltpu.VMEM((g,), jnp.float32),
        ],
    )(acc2, acc2, y2, dinv, b2, batch_p)

    return out
